# baseline (speedup 1.0000x reference)
import functools

import jax
import jax.numpy as jnp
from jax import lax
from jax.experimental import pallas as pl
from jax.experimental.pallas import tpu as pltpu
from jax.experimental.pallas import tpu_sc as plsc

NUM_ANCHORS = 16
K = 10
FEATURE_DIM = 128
ROW_TILE = 1280

NUM_WORKERS = 32
SC_CHUNK = 640


def _head_kernel(x_ref, wb_ref, bb_ref, cw_ref, cb_ref,
                 bboxt_ref, codet_ref, valst_ref):
    m_rows = x_ref.shape[0]
    x = x_ref[:]
    bbox_t = lax.dot_general(wb_ref[:], x, (((0,), (1,)), ((), ())),
                             preferred_element_type=jnp.float32) + bb_ref[:]
    logits_t = lax.dot_general(cw_ref[:], x, (((0,), (1,)), ((), ())),
                               preferred_element_type=jnp.float32) + cb_ref[:]
    c = jax.nn.sigmoid(logits_t)

    iota_a = lax.broadcasted_iota(jnp.int32, (NUM_ANCHORS, m_rows), 0)
    vals_rows, code_rows = [], []
    for _ in range(K):
        m = jnp.max(c, axis=0, keepdims=True)
        idxk = jnp.min(jnp.where(c == m, iota_a, NUM_ANCHORS),
                       axis=0, keepdims=True)
        vals_rows.append(m)
        code_rows.append(jnp.where(m > 0.5, 4 * idxk, -1))
        c = jnp.where(iota_a == idxk, -1.0, c)

    vpad = jnp.zeros((NUM_ANCHORS - K, m_rows), jnp.float32)
    cpad = jnp.full((NUM_ANCHORS - K, m_rows), -1, jnp.int32)
    bboxt_ref[:] = bbox_t
    codet_ref[:] = jnp.concatenate(code_rows + [cpad], axis=0)
    valst_ref[:] = jnp.concatenate(vals_rows + [vpad], axis=0)


def _tc_stage(x, bbox_W, bbox_b, conf_W, conf_b):
    R = x.shape[0]
    bbT = bbox_b[:, None]
    cbT = conf_b[:, None]
    grid = (R // ROW_TILE,)
    return pl.pallas_call(
        _head_kernel,
        grid=grid,
        in_specs=[
            pl.BlockSpec((ROW_TILE, FEATURE_DIM), lambda i: (i, 0)),
            pl.BlockSpec((FEATURE_DIM, NUM_ANCHORS * 4), lambda i: (0, 0)),
            pl.BlockSpec((NUM_ANCHORS * 4, 1), lambda i: (0, 0)),
            pl.BlockSpec((FEATURE_DIM, NUM_ANCHORS), lambda i: (0, 0)),
            pl.BlockSpec((NUM_ANCHORS, 1), lambda i: (0, 0)),
        ],
        out_specs=[
            pl.BlockSpec((NUM_ANCHORS * 4, ROW_TILE), lambda i: (0, i)),
            pl.BlockSpec((NUM_ANCHORS, ROW_TILE), lambda i: (0, i)),
            pl.BlockSpec((NUM_ANCHORS, ROW_TILE), lambda i: (0, i)),
        ],
        out_shape=[
            jax.ShapeDtypeStruct((NUM_ANCHORS * 4, R), jnp.float32),
            jax.ShapeDtypeStruct((NUM_ANCHORS, R), jnp.int32),
            jax.ShapeDtypeStruct((NUM_ANCHORS, R), jnp.float32),
        ],
        compiler_params=pltpu.CompilerParams(
            dimension_semantics=("parallel",),
        ),
    )(x, bbox_W, bbT, conf_W, cbT)


def _sc_stage(codet, valst, bboxt, R):
    C = SC_CHUNK
    n_chunks = R // C
    mesh = plsc.VectorSubcoreMesh(core_axis_name="c", subcore_axis_name="s")

    @functools.partial(
        pl.kernel,
        out_type=[
            jax.ShapeDtypeStruct((R * K * 4,), jnp.float32),
            jax.ShapeDtypeStruct((R * K,), jnp.float32),
        ],
        mesh=mesh,
        scratch_types=[
            pltpu.VMEM((NUM_ANCHORS, C), jnp.int32),
            pltpu.VMEM((NUM_ANCHORS, C), jnp.float32),
            pltpu.VMEM((NUM_ANCHORS * 4 + 8, C), jnp.float32),
            pltpu.VMEM((C * 40 + 16,), jnp.float32),
            pltpu.VMEM((C * 10 + 16,), jnp.float32),
        ],
        compiler_params=pltpu.CompilerParams(needs_layout_passes=False),
    )
    def sc_select(codet_hbm, valst_hbm, bboxt_hbm, det_hbm, vals_hbm,
                  codet_v, valst_v, bboxt_v, det_v, vals_v):
        wid = lax.axis_index("s") * 2 + lax.axis_index("c")
        lane = lax.iota(jnp.int32, 16)
        cmod = lane & 3
        kbase = lane >> 2
        zeros16 = jnp.zeros((16,), jnp.float32)

        def zero_body(o, _):
            bboxt_v[64, pl.ds(o * 16, 16)] = zeros16
            return 0
        lax.fori_loop(0, C // 16, zero_body, 0)

        def chunk_body(ci, _):
            cid = wid + NUM_WORKERS * ci

            @pl.when(cid < n_chunks)
            def _():
                r0 = cid * C
                pltpu.sync_copy(codet_hbm.at[:, pl.ds(r0, C)], codet_v)
                pltpu.sync_copy(valst_hbm.at[:, pl.ds(r0, C)], valst_v)
                pltpu.sync_copy(bboxt_hbm.at[:, pl.ds(r0, C)],
                                bboxt_v.at[pl.ds(0, NUM_ANCHORS * 4), :])

                def row_body(r, _):
                    rvec = jnp.full((16,), 0, jnp.int32) + r
                    vals_g = plsc.load_gather(valst_v, [lane, rvec])
                    vals_v[pl.ds(r * 10, 16)] = vals_g
                    for j in range(3):
                        codes = plsc.load_gather(codet_v, [kbase + 4 * j, rvec])
                        srow = jnp.where(codes >= 0, codes + cmod, 64)
                        det_v[pl.ds(r * 40 + 16 * j, 16)] = (
                            plsc.load_gather(bboxt_v, [srow, rvec]))
                    return 0

                lax.fori_loop(0, C, row_body, 0)
                pltpu.sync_copy(det_v.at[pl.ds(0, C * 40)],
                                det_hbm.at[pl.ds(r0 * 40, C * 40)])
                pltpu.sync_copy(vals_v.at[pl.ds(0, C * 10)],
                                vals_hbm.at[pl.ds(r0 * 10, C * 10)])

            return 0

        n_rounds = (n_chunks + NUM_WORKERS - 1) // NUM_WORKERS
        lax.fori_loop(0, n_rounds, chunk_body, 0)

    return sc_select(codet, valst, bboxt)


@functools.partial(jax.jit, static_argnames=())
def kernel(features, bbox_W, bbox_b, conf_W, conf_b):
    B, T, F = features.shape
    R = B * T
    x = features.reshape(R, F)
    bboxt, codet, valst = _tc_stage(x, bbox_W, bbox_b, conf_W, conf_b)
    det_flat, vals_flat = _sc_stage(codet, valst, bboxt, R)
    return (det_flat.reshape(B, T, K, 4), vals_flat.reshape(B, T, K),
            (vals_flat > 0.5).reshape(B, T, K))


# speedup vs baseline: 1.0561x; 1.0561x over previous
import functools

import jax
import jax.numpy as jnp
from jax import lax
from jax.experimental import pallas as pl
from jax.experimental.pallas import tpu as pltpu
from jax.experimental.pallas import tpu_sc as plsc

NUM_ANCHORS = 16
K = 10
FEATURE_DIM = 128
ROW_TILE = 1280

NUM_WORKERS = 32
SC_CHUNK = 640


def _head_kernel(x_ref, wb_ref, bb_ref, cw_ref, cb_ref,
                 bboxp_ref, codet_ref, valst_ref):
    m_rows = x_ref.shape[0]
    half = m_rows // 2
    x = x_ref[:]
    bbox = jnp.dot(x, wb_ref[:], preferred_element_type=jnp.float32) + bb_ref[:]
    logits_t = lax.dot_general(cw_ref[:], x, (((0,), (1,)), ((), ())),
                               preferred_element_type=jnp.float32) + cb_ref[:]
    c = jax.nn.sigmoid(logits_t)

    iota_a = lax.broadcasted_iota(jnp.int32, (NUM_ANCHORS, m_rows), 0)
    vals_rows, code_rows = [], []
    for _ in range(K):
        m = jnp.max(c, axis=0, keepdims=True)
        idxk = jnp.min(jnp.where(c == m, iota_a, NUM_ANCHORS),
                       axis=0, keepdims=True)
        vals_rows.append(m)
        code_rows.append(jnp.where(m > 0.5, 4 * idxk, -1))
        c = jnp.where(iota_a == idxk, -1.0, c)

    vpad = jnp.zeros((NUM_ANCHORS - K, m_rows), jnp.float32)
    cpad = jnp.full((NUM_ANCHORS - K, m_rows), -1, jnp.int32)
    bboxp_ref[:, : NUM_ANCHORS * 4] = bbox[:half]
    bboxp_ref[:, NUM_ANCHORS * 4 :] = bbox[half:]
    codet_ref[:] = jnp.concatenate(code_rows + [cpad], axis=0)
    valst_ref[:] = jnp.concatenate(vals_rows + [vpad], axis=0)


def _tc_stage(x, bbox_W, bbox_b, conf_W, conf_b):
    R = x.shape[0]
    bb = bbox_b[None, :]
    cbT = conf_b[:, None]
    grid = (R // ROW_TILE,)
    return pl.pallas_call(
        _head_kernel,
        grid=grid,
        in_specs=[
            pl.BlockSpec((ROW_TILE, FEATURE_DIM), lambda i: (i, 0)),
            pl.BlockSpec((FEATURE_DIM, NUM_ANCHORS * 4), lambda i: (0, 0)),
            pl.BlockSpec((1, NUM_ANCHORS * 4), lambda i: (0, 0)),
            pl.BlockSpec((FEATURE_DIM, NUM_ANCHORS), lambda i: (0, 0)),
            pl.BlockSpec((NUM_ANCHORS, 1), lambda i: (0, 0)),
        ],
        out_specs=[
            pl.BlockSpec((ROW_TILE // 2, 2 * NUM_ANCHORS * 4), lambda i: (i, 0)),
            pl.BlockSpec((NUM_ANCHORS, ROW_TILE), lambda i: (0, i)),
            pl.BlockSpec((NUM_ANCHORS, ROW_TILE), lambda i: (0, i)),
        ],
        out_shape=[
            jax.ShapeDtypeStruct((R // 2, 2 * NUM_ANCHORS * 4), jnp.float32),
            jax.ShapeDtypeStruct((NUM_ANCHORS, R), jnp.int32),
            jax.ShapeDtypeStruct((NUM_ANCHORS, R), jnp.float32),
        ],
        compiler_params=pltpu.CompilerParams(
            dimension_semantics=("parallel",),
        ),
    )(x, bbox_W, bb, conf_W, cbT)


def _sc_stage(codet, valst, bboxp, R):
    C = SC_CHUNK
    n_chunks = R // C
    mesh = plsc.VectorSubcoreMesh(core_axis_name="c", subcore_axis_name="s")

    @functools.partial(
        pl.kernel,
        out_type=[
            jax.ShapeDtypeStruct((R * K * 4,), jnp.float32),
            jax.ShapeDtypeStruct((R * K,), jnp.float32),
        ],
        mesh=mesh,
        scratch_types=[
            pltpu.VMEM((NUM_ANCHORS, C), jnp.int32),
            pltpu.VMEM((NUM_ANCHORS, C), jnp.float32),
            pltpu.VMEM((C + 8, 2 * NUM_ANCHORS * 4), jnp.float32),
            pltpu.VMEM((C // 2 * 40 + 16,), jnp.float32),
            pltpu.VMEM((C * 10 + 16,), jnp.float32),
        ],
        compiler_params=pltpu.CompilerParams(needs_layout_passes=False),
    )
    def sc_select(codet_hbm, valst_hbm, bboxp_hbm, det_hbm, vals_hbm,
                  codet_v, valst_v, bboxp_v, det_v, vals_v):
        wid = lax.axis_index("s") * 2 + lax.axis_index("c")
        lane = lax.iota(jnp.int32, 16)
        cmod = lane & 3
        kbase = lane >> 2
        zeros16 = jnp.zeros((16,), jnp.float32)

        for o in range(8):
            bboxp_v[C, pl.ds(o * 16, 16)] = zeros16

        def chunk_body(ci, _):
            cid = wid + NUM_WORKERS * ci

            @pl.when(cid < n_chunks)
            def _():
                r0 = cid * C
                half = (cid & 1) * (NUM_ANCHORS * 4)
                prow = (cid >> 1) * C
                pltpu.sync_copy(codet_hbm.at[:, pl.ds(r0, C)], codet_v)
                pltpu.sync_copy(valst_hbm.at[:, pl.ds(r0, C)], valst_v)
                pltpu.sync_copy(bboxp_hbm.at[pl.ds(prow, C), :],
                                bboxp_v.at[pl.ds(0, C), :])

                for q in range(2):
                    def row_body(r, _):
                        rr = q * (C // 2) + r
                        rvec = jnp.full((16,), 0, jnp.int32) + rr
                        vals_g = plsc.load_gather(valst_v, [lane, rvec])
                        vals_v[pl.ds(rr * 10, 16)] = vals_g
                        for j in range(3):
                            codes = plsc.load_gather(codet_v,
                                                     [kbase + 4 * j, rvec])
                            ok = codes >= 0
                            srow = jnp.where(ok, rvec, C)
                            scol = jnp.where(ok, codes, 0) + cmod + half
                            det_v[pl.ds(r * 40 + 16 * j, 16)] = (
                                plsc.load_gather(bboxp_v, [srow, scol]))
                        return 0

                    lax.fori_loop(0, C // 2, row_body, 0)
                    pltpu.sync_copy(
                        det_v.at[pl.ds(0, C // 2 * 40)],
                        det_hbm.at[pl.ds((r0 + q * (C // 2)) * 40, C // 2 * 40)])
                pltpu.sync_copy(vals_v.at[pl.ds(0, C * 10)],
                                vals_hbm.at[pl.ds(r0 * 10, C * 10)])

            return 0

        n_rounds = (n_chunks + NUM_WORKERS - 1) // NUM_WORKERS
        lax.fori_loop(0, n_rounds, chunk_body, 0)

    return sc_select(codet, valst, bboxp)


@functools.partial(jax.jit, static_argnames=())
def kernel(features, bbox_W, bbox_b, conf_W, conf_b):
    B, T, F = features.shape
    R = B * T
    x = features.reshape(R, F)
    bboxp, codet, valst = _tc_stage(x, bbox_W, bbox_b, conf_W, conf_b)
    det_flat, vals_flat = _sc_stage(codet, valst, bboxp, R)
    return (det_flat.reshape(B, T, K, 4), vals_flat.reshape(B, T, K),
            (vals_flat > 0.5).reshape(B, T, K))
